# baseline (speedup 1.0000x reference)
import functools

import jax
import jax.numpy as jnp
from jax import lax
from jax.experimental import pallas as pl
from jax.experimental.pallas import tpu as pltpu
from jax.experimental.pallas import tpu_sc as plsc

N_NODES = 50000
HIDDEN = 512
NC = 2
NS = 16
NW = NC * NS
C = 40
SZ_BIG = 1600
SZ_SML = 1560
VPR = HIDDEN // 16


def _ce_body(in_idx, out_idx, in_tab, out_tab, out,
             idx_in, idx_out, buf_in0, buf_out0, buf_in1, buf_out1,
             sem_in0, sem_out0, sem_in1, sem_out1, sem_st0, sem_st1):
  wid = lax.axis_index("s") * NC + lax.axis_index("c")
  base = wid * SZ_SML + jnp.minimum(wid, 2) * (SZ_BIG - SZ_SML)
  nw = jnp.where(wid < 2, SZ_BIG // C, SZ_SML // C)

  @pl.when(wid < 2)
  def _():
    pltpu.sync_copy(in_idx.at[pl.ds(base, SZ_BIG)], idx_in)
    pltpu.sync_copy(out_idx.at[pl.ds(base, SZ_BIG)], idx_out)

  @pl.when(wid >= 2)
  def _():
    pltpu.sync_copy(in_idx.at[pl.ds(base, SZ_SML)], idx_in.at[pl.ds(0, SZ_SML)])
    pltpu.sync_copy(out_idx.at[pl.ds(base, SZ_SML)],
                    idx_out.at[pl.ds(0, SZ_SML)])

  bufs = ((buf_in0, buf_out0, sem_in0, sem_out0, sem_st0),
          (buf_in1, buf_out1, sem_in1, sem_out1, sem_st1))

  def issue(j, slot):
    b_in, b_out, s_in, s_out, s_st = bufs[slot]

    @pl.when(j < nw)
    def _():
      @pl.when(j >= 2)
      def _():
        pltpu.make_async_copy(b_in, out.at[pl.ds(base + (j - 2) * C, C)],
                              s_st).wait()

      pltpu.async_copy(in_tab.at[idx_in.at[pl.ds(j * C, C)]], b_in, s_in)
      pltpu.async_copy(out_tab.at[idx_out.at[pl.ds(j * C, C)]], b_out, s_out)

  def process(j, slot):
    b_in, b_out, s_in, s_out, s_st = bufs[slot]

    @pl.when(j < nw)
    def _():
      pltpu.make_async_copy(in_tab.at[idx_in.at[pl.ds(j * C, C)]], b_in,
                            s_in).wait()
      pltpu.make_async_copy(out_tab.at[idx_out.at[pl.ds(j * C, C)]], b_out,
                            s_out).wait()

      def add_row(r, _):
        for k in range(VPR):
          sl = pl.ds(k * 16, 16)
          plsc.addupdate(b_in.at[r, sl], b_out[r, sl])
        return 0

      lax.fori_loop(0, C, add_row, 0)
      pltpu.async_copy(b_in, out.at[pl.ds(base + j * C, C)], s_st)

  issue(0, 0)

  def group(g, _):
    for b in range(2):
      j = g * 2 + b
      issue(j + 1, 1 - b)
      process(j, b)
    return 0

  lax.fori_loop(0, (nw + 1) // 2, group, 0)

  for s in range(2):
    b_in = bufs[s][0]
    s_st = bufs[s][4]
    j_last = jnp.where(((nw - 1) % 2) == s, nw - 1, nw - 2)
    pltpu.make_async_copy(b_in, out.at[pl.ds(base + j_last * C, C)],
                          s_st).wait()


@jax.jit
def kernel(in_degree_list, out_degree_list, in_table, out_table):
  mesh = plsc.VectorSubcoreMesh(core_axis_name="c", subcore_axis_name="s")
  f = functools.partial(
      pl.kernel,
      out_type=jax.ShapeDtypeStruct((N_NODES, HIDDEN), jnp.float32),
      mesh=mesh,
      scratch_types=[
          pltpu.VMEM((SZ_BIG,), jnp.int32),
          pltpu.VMEM((SZ_BIG,), jnp.int32),
          pltpu.VMEM((C, HIDDEN), jnp.float32),
          pltpu.VMEM((C, HIDDEN), jnp.float32),
          pltpu.VMEM((C, HIDDEN), jnp.float32),
          pltpu.VMEM((C, HIDDEN), jnp.float32),
          pltpu.SemaphoreType.DMA,
          pltpu.SemaphoreType.DMA,
          pltpu.SemaphoreType.DMA,
          pltpu.SemaphoreType.DMA,
          pltpu.SemaphoreType.DMA,
          pltpu.SemaphoreType.DMA,
      ],
  )(_ce_body)
  return f(in_degree_list.astype(jnp.int32), out_degree_list.astype(jnp.int32),
           in_table, out_table)


# speedup vs baseline: 1.0206x; 1.0206x over previous
import functools

import jax
import jax.numpy as jnp
from jax import lax
from jax.experimental import pallas as pl
from jax.experimental.pallas import tpu as pltpu
from jax.experimental.pallas import tpu_sc as plsc

N_NODES = 50000
HIDDEN = 512
NC = 2
NS = 16
NW = NC * NS
C = 40
SZ_BIG = 1600
SZ_SML = 1560
VPR = HIDDEN // 16


def _ce_body(in_idx, out_idx, in_tab, out_tab, out,
             idx_in, idx_out, buf_in0, buf_out0, buf_in1, buf_out1, sp_stage,
             sem_in0, sem_out0, sem_in1, sem_out1, sem_st0, sem_st1):
  sid = lax.axis_index("s")
  wid = sid * NC + lax.axis_index("c")
  base = wid * SZ_SML + jnp.minimum(wid, 2) * (SZ_BIG - SZ_SML)
  nw = jnp.where(wid < 2, SZ_BIG // C, SZ_SML // C)

  @pl.when(wid < 2)
  def _():
    pltpu.sync_copy(in_idx.at[pl.ds(base, SZ_BIG)], idx_in)
    pltpu.sync_copy(out_idx.at[pl.ds(base, SZ_BIG)], idx_out)

  @pl.when(wid >= 2)
  def _():
    pltpu.sync_copy(in_idx.at[pl.ds(base, SZ_SML)], idx_in.at[pl.ds(0, SZ_SML)])
    pltpu.sync_copy(out_idx.at[pl.ds(base, SZ_SML)],
                    idx_out.at[pl.ds(0, SZ_SML)])

  bufs = ((buf_in0, buf_out0, sem_in0, sem_out0, sem_st0),
          (buf_in1, buf_out1, sem_in1, sem_out1, sem_st1))

  def issue(j, slot):
    b_in, b_out, s_in, s_out, _ = bufs[slot]

    @pl.when(j < nw)
    def _():
      pltpu.async_copy(in_tab.at[idx_in.at[pl.ds(j * C, C)]], b_in, s_in)
      pltpu.async_copy(out_tab.at[idx_out.at[pl.ds(j * C, C)]], b_out, s_out)

  def process(j, slot):
    b_in, b_out, s_in, s_out, s_st = bufs[slot]
    stage = sp_stage.at[sid, slot]

    @pl.when(j < nw)
    def _():
      pltpu.make_async_copy(in_tab.at[idx_in.at[pl.ds(j * C, C)]], b_in,
                            s_in).wait()
      pltpu.make_async_copy(out_tab.at[idx_out.at[pl.ds(j * C, C)]], b_out,
                            s_out).wait()

      def add_row(r, _):
        for k in range(VPR):
          sl = pl.ds(k * 16, 16)
          plsc.addupdate(b_in.at[r, sl], b_out[r, sl])
        return 0

      lax.fori_loop(0, C, add_row, 0)

      @pl.when(j >= 2)
      def _():
        pltpu.make_async_copy(stage, out.at[pl.ds(base + (j - 2) * C, C)],
                              s_st).wait()

      pltpu.sync_copy(b_in, stage)
      pltpu.async_copy(stage, out.at[pl.ds(base + j * C, C)], s_st)

  issue(0, 0)

  def group(g, _):
    for b in range(2):
      j = g * 2 + b
      issue(j + 1, 1 - b)
      process(j, b)
    return 0

  lax.fori_loop(0, (nw + 1) // 2, group, 0)

  for s in range(2):
    s_st = bufs[s][4]
    j_last = jnp.where(((nw - 1) % 2) == s, nw - 1, nw - 2)
    pltpu.make_async_copy(sp_stage.at[sid, s],
                          out.at[pl.ds(base + j_last * C, C)], s_st).wait()


@jax.jit
def kernel(in_degree_list, out_degree_list, in_table, out_table):
  mesh = plsc.VectorSubcoreMesh(core_axis_name="c", subcore_axis_name="s")
  f = functools.partial(
      pl.kernel,
      out_type=jax.ShapeDtypeStruct((N_NODES, HIDDEN), jnp.float32),
      mesh=mesh,
      scratch_types=[
          pltpu.VMEM((SZ_BIG,), jnp.int32),
          pltpu.VMEM((SZ_BIG,), jnp.int32),
          pltpu.VMEM((C, HIDDEN), jnp.float32),
          pltpu.VMEM((C, HIDDEN), jnp.float32),
          pltpu.VMEM((C, HIDDEN), jnp.float32),
          pltpu.VMEM((C, HIDDEN), jnp.float32),
          pltpu.VMEM_SHARED((NS, 2, C, HIDDEN), jnp.float32),
          pltpu.SemaphoreType.DMA,
          pltpu.SemaphoreType.DMA,
          pltpu.SemaphoreType.DMA,
          pltpu.SemaphoreType.DMA,
          pltpu.SemaphoreType.DMA,
          pltpu.SemaphoreType.DMA,
      ],
  )(_ce_body)
  return f(in_degree_list.astype(jnp.int32), out_degree_list.astype(jnp.int32),
           in_table, out_table)
